# baseline (speedup 1.0000x reference)
import functools

import jax
import jax.numpy as jnp
from jax import lax
from jax.experimental import pallas as pl
from jax.experimental.pallas import tpu as pltpu
from jax.experimental.pallas import tpu_sc as plsc

N = 10000
E = 320000
D = 128
NL = 4
NCLS = 10

NCORE = 2
NSUB = 16
NW = NCORE * NSUB
C = 128
NCHUNK = 80
EPW = NCHUNK * C
EP = EPW * NW
NP = 10240
DUMMY = N + 16
SLAB = NP // NSUB
NH = NCHUNK // 2
CA = 32
CB = 2 * NCHUNK - CA


@functools.cache
def _mesh():
    return plsc.VectorSubcoreMesh(core_axis_name="c", subcore_axis_name="s",
                                  num_cores=NCORE, num_subcores=NSUB)


def _fill_cols(ref, nrows, val_left, val_right):
    vl = jnp.full((16,), val_left, jnp.float32)
    vr = jnp.full((16,), val_right, jnp.float32)

    def body(i, carry):
        for j in range(8):
            ref[i, pl.ds(j * 16, 16)] = vl if j < 4 else vr
        return carry

    lax.fori_loop(0, nrows, body, 0)


@functools.cache
def _deg_kernel():
    return functools.partial(
        pl.kernel,
        out_type=jax.ShapeDtypeStruct((NCORE, NP, D), jnp.float32),
        mesh=_mesh(),
        scratch_types=[
            pltpu.VMEM_SHARED((NP, D), jnp.float32),
            pltpu.VMEM((C, D), jnp.float32),
            pltpu.VMEM((C, D), jnp.float32),
            pltpu.VMEM((NH, C), jnp.int32),
            pltpu.VMEM((NH, C), jnp.int32),
        ],
    )(_deg_body)


def _deg_body(srcp, dstp, out, acc, ones_s, ones_d, sidx, didx):
    c = lax.axis_index("c")
    s = lax.axis_index("s")
    wid = s * NCORE + c

    pltpu.sync_copy(srcp.at[pl.ds(wid * NCHUNK, NH)], sidx)
    pltpu.sync_copy(dstp.at[pl.ds(wid * NCHUNK, NH)], didx)
    _fill_cols(ones_d, C, 0.0, 0.0)
    for k in range(SLAB // C):
        pltpu.sync_copy(ones_d, acc.at[pl.ds(s * SLAB + k * C, C)])
    _fill_cols(ones_s, C, 1.0, 0.0)
    _fill_cols(ones_d, C, 0.0, 1.0)
    plsc.subcore_barrier()

    def ebody(g, carry):
        pltpu.sync_copy(ones_s, acc.at[sidx.at[g]], add=True)
        pltpu.sync_copy(ones_d, acc.at[didx.at[g]], add=True)
        return carry

    lax.fori_loop(0, NH, ebody, 0)
    pltpu.sync_copy(srcp.at[pl.ds(wid * NCHUNK + NH, NH)], sidx)
    pltpu.sync_copy(dstp.at[pl.ds(wid * NCHUNK + NH, NH)], didx)
    lax.fori_loop(0, NH, ebody, 0)
    plsc.subcore_barrier()

    for k in range(SLAB // C):
        sl = pl.ds(s * SLAB + k * C, C)
        pltpu.sync_copy(acc.at[sl], out.at[c, sl])


@functools.cache
def _seg_kernel():
    return functools.partial(
        pl.kernel,
        out_type=jax.ShapeDtypeStruct((NCORE, NP, D), jnp.float32),
        mesh=_mesh(),
        scratch_types=[
            pltpu.VMEM_SHARED((NP, D), jnp.float32),
            pltpu.VMEM((C, D), jnp.float32),
            pltpu.VMEM((C, D), jnp.float32),
            pltpu.VMEM((NH, C), jnp.int32),
            pltpu.VMEM((NH, C), jnp.int32),
            pltpu.SemaphoreType.DMA,
            pltpu.SemaphoreType.DMA,
        ],
    )(_seg_body)


def _seg_body(hx, srcp, dstp, out, acc, rows0, rows1, sidx, didx,
              sem0, sem1):
    c = lax.axis_index("c")
    s = lax.axis_index("s")
    wid = s * NCORE + c

    _fill_cols(rows0, C, 0.0, 0.0)
    for k in range(SLAB // C):
        pltpu.sync_copy(rows0, acc.at[pl.ds(s * SLAB + k * C, C)])
    plsc.subcore_barrier()

    def _run(base_row, count):
        def ebody(n):
            def body(q, carry):
                g = q * 2
                pltpu.async_copy(hx.at[sidx.at[g + 1]], rows1, sem1)
                pltpu.make_async_copy(hx.at[sidx.at[g]], rows0, sem0).wait()
                pltpu.sync_copy(rows0, acc.at[didx.at[g]], add=True)
                gnext = jnp.minimum(g + 2, n - 1)
                pltpu.async_copy(hx.at[sidx.at[gnext]], rows0, sem0)
                pltpu.make_async_copy(hx.at[sidx.at[g + 1]], rows1, sem1).wait()
                pltpu.sync_copy(rows1, acc.at[didx.at[g + 1]], add=True)
                return carry
            return body

        done = 0
        while done < count:
            n = min(NH, count - done)
            pltpu.sync_copy(srcp.at[pl.ds(base_row + done, n)], sidx.at[pl.ds(0, n)])
            pltpu.sync_copy(dstp.at[pl.ds(base_row + done, n)], didx.at[pl.ds(0, n)])
            pltpu.async_copy(hx.at[sidx.at[0]], rows0, sem0)
            lax.fori_loop(0, n // 2, ebody(n), 0)
            pltpu.make_async_copy(hx.at[sidx.at[n - 1]], rows0, sem0).wait()
            done += n

    pair_base = s * (2 * NCHUNK)

    @pl.when(c == 0)
    def _():
        _run(pair_base, CA)

    @pl.when(c == 1)
    def _():
        _run(pair_base + CA, CB)

    plsc.subcore_barrier()

    for k in range(SLAB // C):
        sl = pl.ds(s * SLAB + k * C, C)
        pltpu.sync_copy(acc.at[sl], out.at[c, sl])


def _norm(dg_ref, which):
    col = 0 if which == 0 else 64
    d = (dg_ref[0, pl.ds(0, N), pl.ds(col, 1)]
         + dg_ref[1, pl.ds(0, N), pl.ds(col, 1)])
    return lax.rsqrt(jnp.maximum(d, 1.0))


BLK = NP // 8


def _mm_body(src_ref, w_ref, b_ref, dst_ref):
    dst_ref[...] = jnp.dot(
        src_ref[...].astype(jnp.bfloat16), w_ref[...].astype(jnp.bfloat16),
        preferred_element_type=jnp.float32) + b_ref[...]


def _mms_body(src_ref, dg_ref, w_ref, dst_ref):
    d = dg_ref[0, :, 0:1] + dg_ref[1, :, 0:1]
    ns = lax.rsqrt(jnp.maximum(d, 1.0))
    dst_ref[...] = jnp.dot(
        (src_ref[...] * ns).astype(jnp.bfloat16),
        w_ref[...].astype(jnp.bfloat16),
        preferred_element_type=jnp.float32)


def _mm(src, w, b):
    return pl.pallas_call(
        _mm_body,
        grid=(NP // BLK,),
        in_specs=[pl.BlockSpec((BLK, D), lambda i: (i, 0)),
                  pl.BlockSpec((D, D), lambda i: (0, 0)),
                  pl.BlockSpec((1, D), lambda i: (0, 0))],
        out_specs=pl.BlockSpec((BLK, D), lambda i: (i, 0)),
        out_shape=_sds((NP, D)),
    )(src, w, b)


def _mms(src, dg, w):
    return pl.pallas_call(
        _mms_body,
        grid=(NP // BLK,),
        in_specs=[pl.BlockSpec((BLK, D), lambda i: (i, 0)),
                  pl.BlockSpec((2, BLK, D), lambda i: (0, i, 0)),
                  pl.BlockSpec((D, D), lambda i: (0, 0))],
        out_specs=pl.BlockSpec((BLK, D), lambda i: (i, 0)),
        out_shape=_sds((NP, D)),
    )(src, dg, w)


def _t1a_body(parts_ref, x_ref, dg_ref, bg_ref, ga_ref, be_ref, xn_ref):
    agg = (parts_ref[0, pl.ds(0, N), :] + parts_ref[1, pl.ds(0, N), :])
    agg = agg * _norm(dg_ref, 1) + bg_ref[...]
    mu = jnp.mean(agg, axis=0, keepdims=True)
    var = jnp.mean((agg - mu) ** 2, axis=0, keepdims=True)
    an = (agg - mu) * lax.rsqrt(var + 1e-5) * ga_ref[...] + be_ref[...]
    xn_ref[pl.ds(0, N), :] = x_ref[pl.ds(0, N), :] + jnp.maximum(an, 0.0)


def _tf_body(parts_ref, x_ref, dg_ref, bg_ref, ga_ref, be_ref,
             w0_ref, b0_ref, w1_ref, b1_ref, w2_ref, b2_ref, out_ref):
    agg = (parts_ref[0, pl.ds(0, N), :] + parts_ref[1, pl.ds(0, N), :])
    agg = agg * _norm(dg_ref, 1) + bg_ref[...]
    mu = jnp.mean(agg, axis=0, keepdims=True)
    var = jnp.mean((agg - mu) ** 2, axis=0, keepdims=True)
    an = (agg - mu) * lax.rsqrt(var + 1e-5) * ga_ref[...] + be_ref[...]
    xn = x_ref[pl.ds(0, N), :] + jnp.maximum(an, 0.0)
    hg = jnp.mean(xn, axis=0, keepdims=True)
    h0 = jnp.maximum(jnp.dot(hg.astype(jnp.bfloat16),
                             w0_ref[...].astype(jnp.bfloat16),
                             preferred_element_type=jnp.float32)
                     + b0_ref[...], 0.0)
    h1 = jnp.maximum(jnp.dot(h0.astype(jnp.bfloat16),
                             w1_ref[...].astype(jnp.bfloat16),
                             preferred_element_type=jnp.float32)
                     + b1_ref[...], 0.0)
    out_ref[...] = jnp.dot(h1.astype(jnp.bfloat16),
                           w2_ref[...].astype(jnp.bfloat16),
                           preferred_element_type=jnp.float32) + b2_ref[...]


def _sds(shape):
    return jax.ShapeDtypeStruct(shape, jnp.float32)


def kernel(h, edge_index, e, W_emb, b_emb, Wg, bg, gamma, beta,
           W0, b0, W1, b1, W2, b2):
    pad = jnp.full((EP - E,), DUMMY, jnp.int32)
    srcp = jnp.concatenate([edge_index[0], pad]).reshape(NW * NCHUNK, C)
    dstp = jnp.concatenate([edge_index[1], pad]).reshape(NW * NCHUNK, C)
    srcg = jnp.where(srcp < N, srcp, N)

    dg = _deg_kernel()(srcp, dstp)

    hp = jnp.concatenate([h, jnp.zeros((NP - N, D), jnp.float32)])
    x = _mm(hp, W_emb, b_emb.reshape(1, D))
    hx = _mms(x, dg, Wg[0])

    out = None
    for l in range(NL):
        parts = _seg_kernel()(hx, srcg, dstp)
        if l < NL - 1:
            x = pl.pallas_call(
                _t1a_body,
                out_shape=_sds((NP, D)),
            )(parts, x, dg, bg[l].reshape(1, D),
              gamma[l].reshape(1, D), beta[l].reshape(1, D))
            hx = _mms(x, dg, Wg[l + 1])
        else:
            out = pl.pallas_call(
                _tf_body,
                out_shape=_sds((1, NCLS)),
            )(parts, x, dg, bg[l].reshape(1, D), gamma[l].reshape(1, D),
              beta[l].reshape(1, D), W0, b0.reshape(1, D // 2),
              W1, b1.reshape(1, D // 4), W2, b2.reshape(1, NCLS))
    return out


# speedup vs baseline: 1.2270x; 1.2270x over previous
import functools

import jax
import jax.numpy as jnp
from jax import lax
from jax.experimental import pallas as pl
from jax.experimental.pallas import tpu as pltpu
from jax.experimental.pallas import tpu_sc as plsc

N = 10000
E = 320000
D = 128
NL = 4
NCLS = 10

NCORE = 2
NSUB = 16
NW = NCORE * NSUB
C = 128
NCHUNK = 80
EPW = NCHUNK * C
EP = EPW * NW
NP = 10240
DUMMY = N + 16
SLAB = NP // NSUB
NH = NCHUNK // 2
CA = 128
CB = 2 * NCHUNK - CA


@functools.cache
def _mesh():
    return plsc.VectorSubcoreMesh(core_axis_name="c", subcore_axis_name="s",
                                  num_cores=NCORE, num_subcores=NSUB)


def _fill_cols(ref, nrows, val_left, val_right):
    vl = jnp.full((16,), val_left, jnp.float32)
    vr = jnp.full((16,), val_right, jnp.float32)

    def body(i, carry):
        for j in range(8):
            ref[i, pl.ds(j * 16, 16)] = vl if j < 4 else vr
        return carry

    lax.fori_loop(0, nrows, body, 0)


@functools.cache
def _deg_kernel():
    return functools.partial(
        pl.kernel,
        out_type=jax.ShapeDtypeStruct((NCORE, NP, D), jnp.float32),
        mesh=_mesh(),
        scratch_types=[
            pltpu.VMEM_SHARED((NP, D), jnp.float32),
            pltpu.VMEM((C, D), jnp.float32),
            pltpu.VMEM((C, D), jnp.float32),
            pltpu.VMEM((NH, C), jnp.int32),
            pltpu.VMEM((NH, C), jnp.int32),
        ],
    )(_deg_body)


def _deg_body(srcp, dstp, out, acc, ones_s, ones_d, sidx, didx):
    c = lax.axis_index("c")
    s = lax.axis_index("s")
    wid = s * NCORE + c

    pltpu.sync_copy(srcp.at[pl.ds(wid * NCHUNK, NH)], sidx)
    pltpu.sync_copy(dstp.at[pl.ds(wid * NCHUNK, NH)], didx)
    _fill_cols(ones_d, C, 0.0, 0.0)
    for k in range(SLAB // C):
        pltpu.sync_copy(ones_d, acc.at[pl.ds(s * SLAB + k * C, C)])
    _fill_cols(ones_s, C, 1.0, 0.0)
    _fill_cols(ones_d, C, 0.0, 1.0)
    plsc.subcore_barrier()

    def ebody(g, carry):
        pltpu.sync_copy(ones_s, acc.at[sidx.at[g]], add=True)
        pltpu.sync_copy(ones_d, acc.at[didx.at[g]], add=True)
        return carry

    lax.fori_loop(0, NH, ebody, 0)
    pltpu.sync_copy(srcp.at[pl.ds(wid * NCHUNK + NH, NH)], sidx)
    pltpu.sync_copy(dstp.at[pl.ds(wid * NCHUNK + NH, NH)], didx)
    lax.fori_loop(0, NH, ebody, 0)
    plsc.subcore_barrier()

    for k in range(SLAB // C):
        sl = pl.ds(s * SLAB + k * C, C)
        pltpu.sync_copy(acc.at[sl], out.at[c, sl])


@functools.cache
def _seg_kernel():
    return functools.partial(
        pl.kernel,
        out_type=jax.ShapeDtypeStruct((NCORE, NP, D), jnp.float32),
        mesh=_mesh(),
        scratch_types=[
            pltpu.VMEM_SHARED((NP, D), jnp.float32),
            pltpu.VMEM((C, D), jnp.float32),
            pltpu.VMEM((C, D), jnp.float32),
            pltpu.VMEM((NH, C), jnp.int32),
            pltpu.VMEM((NH, C), jnp.int32),
            pltpu.SemaphoreType.DMA,
            pltpu.SemaphoreType.DMA,
        ],
    )(_seg_body)


def _seg_body(hx, srcp, dstp, out, acc, rows0, rows1, sidx, didx,
              sem0, sem1):
    c = lax.axis_index("c")
    s = lax.axis_index("s")
    wid = s * NCORE + c

    _fill_cols(rows0, C, 0.0, 0.0)
    for k in range(SLAB // C):
        pltpu.sync_copy(rows0, acc.at[pl.ds(s * SLAB + k * C, C)])
    plsc.subcore_barrier()

    def _run(base_row, count):
        def ebody(n):
            def body(q, carry):
                g = q * 2
                pltpu.async_copy(hx.at[sidx.at[g + 1]], rows1, sem1)
                pltpu.make_async_copy(hx.at[sidx.at[g]], rows0, sem0).wait()
                pltpu.sync_copy(rows0, acc.at[didx.at[g]], add=True)
                gnext = jnp.minimum(g + 2, n - 1)
                pltpu.async_copy(hx.at[sidx.at[gnext]], rows0, sem0)
                pltpu.make_async_copy(hx.at[sidx.at[g + 1]], rows1, sem1).wait()
                pltpu.sync_copy(rows1, acc.at[didx.at[g + 1]], add=True)
                return carry
            return body

        done = 0
        while done < count:
            n = min(NH, count - done)
            pltpu.sync_copy(srcp.at[pl.ds(base_row + done, n)], sidx.at[pl.ds(0, n)])
            pltpu.sync_copy(dstp.at[pl.ds(base_row + done, n)], didx.at[pl.ds(0, n)])
            pltpu.async_copy(hx.at[sidx.at[0]], rows0, sem0)
            lax.fori_loop(0, n // 2, ebody(n), 0)
            pltpu.make_async_copy(hx.at[sidx.at[n - 1]], rows0, sem0).wait()
            done += n

    pair_base = s * (2 * NCHUNK)

    @pl.when(c == 0)
    def _():
        _run(pair_base, CA)

    @pl.when(c == 1)
    def _():
        _run(pair_base + CA, CB)

    plsc.subcore_barrier()

    for k in range(SLAB // C):
        sl = pl.ds(s * SLAB + k * C, C)
        pltpu.sync_copy(acc.at[sl], out.at[c, sl])


def _norm(dg_ref, which):
    col = 0 if which == 0 else 64
    d = (dg_ref[0, pl.ds(0, N), pl.ds(col, 1)]
         + dg_ref[1, pl.ds(0, N), pl.ds(col, 1)])
    return lax.rsqrt(jnp.maximum(d, 1.0))


BLK = NP // 8


def _mm_body(src_ref, w_ref, b_ref, dst_ref):
    dst_ref[...] = jnp.dot(
        src_ref[...].astype(jnp.bfloat16), w_ref[...].astype(jnp.bfloat16),
        preferred_element_type=jnp.float32) + b_ref[...]


def _mms_body(src_ref, dg_ref, w_ref, dst_ref):
    d = dg_ref[0, :, 0:1] + dg_ref[1, :, 0:1]
    ns = lax.rsqrt(jnp.maximum(d, 1.0))
    dst_ref[...] = jnp.dot(
        (src_ref[...] * ns).astype(jnp.bfloat16),
        w_ref[...].astype(jnp.bfloat16),
        preferred_element_type=jnp.float32)


def _mm(src, w, b):
    return pl.pallas_call(
        _mm_body,
        grid=(NP // BLK,),
        in_specs=[pl.BlockSpec((BLK, D), lambda i: (i, 0)),
                  pl.BlockSpec((D, D), lambda i: (0, 0)),
                  pl.BlockSpec((1, D), lambda i: (0, 0))],
        out_specs=pl.BlockSpec((BLK, D), lambda i: (i, 0)),
        out_shape=_sds((NP, D)),
    )(src, w, b)


def _mms(src, dg, w):
    return pl.pallas_call(
        _mms_body,
        grid=(NP // BLK,),
        in_specs=[pl.BlockSpec((BLK, D), lambda i: (i, 0)),
                  pl.BlockSpec((2, BLK, D), lambda i: (0, i, 0)),
                  pl.BlockSpec((D, D), lambda i: (0, 0))],
        out_specs=pl.BlockSpec((BLK, D), lambda i: (i, 0)),
        out_shape=_sds((NP, D)),
    )(src, dg, w)


def _t1a_body(parts_ref, x_ref, dg_ref, bg_ref, ga_ref, be_ref, xn_ref):
    agg = (parts_ref[0, pl.ds(0, N), :] + parts_ref[1, pl.ds(0, N), :])
    agg = agg * _norm(dg_ref, 1) + bg_ref[...]
    mu = jnp.mean(agg, axis=0, keepdims=True)
    var = jnp.mean((agg - mu) ** 2, axis=0, keepdims=True)
    an = (agg - mu) * lax.rsqrt(var + 1e-5) * ga_ref[...] + be_ref[...]
    xn_ref[pl.ds(0, N), :] = x_ref[pl.ds(0, N), :] + jnp.maximum(an, 0.0)


def _tf_body(parts_ref, x_ref, dg_ref, bg_ref, ga_ref, be_ref,
             w0_ref, b0_ref, w1_ref, b1_ref, w2_ref, b2_ref, out_ref):
    agg = (parts_ref[0, pl.ds(0, N), :] + parts_ref[1, pl.ds(0, N), :])
    agg = agg * _norm(dg_ref, 1) + bg_ref[...]
    mu = jnp.mean(agg, axis=0, keepdims=True)
    var = jnp.mean((agg - mu) ** 2, axis=0, keepdims=True)
    an = (agg - mu) * lax.rsqrt(var + 1e-5) * ga_ref[...] + be_ref[...]
    xn = x_ref[pl.ds(0, N), :] + jnp.maximum(an, 0.0)
    hg = jnp.mean(xn, axis=0, keepdims=True)
    h0 = jnp.maximum(jnp.dot(hg.astype(jnp.bfloat16),
                             w0_ref[...].astype(jnp.bfloat16),
                             preferred_element_type=jnp.float32)
                     + b0_ref[...], 0.0)
    h1 = jnp.maximum(jnp.dot(h0.astype(jnp.bfloat16),
                             w1_ref[...].astype(jnp.bfloat16),
                             preferred_element_type=jnp.float32)
                     + b1_ref[...], 0.0)
    out_ref[...] = jnp.dot(h1.astype(jnp.bfloat16),
                           w2_ref[...].astype(jnp.bfloat16),
                           preferred_element_type=jnp.float32) + b2_ref[...]


def _sds(shape):
    return jax.ShapeDtypeStruct(shape, jnp.float32)


def kernel(h, edge_index, e, W_emb, b_emb, Wg, bg, gamma, beta,
           W0, b0, W1, b1, W2, b2):
    pad = jnp.full((EP - E,), DUMMY, jnp.int32)
    srcp = jnp.concatenate([edge_index[0], pad]).reshape(NW * NCHUNK, C)
    dstp = jnp.concatenate([edge_index[1], pad]).reshape(NW * NCHUNK, C)
    srcg = jnp.where(srcp < N, srcp, N)

    dg = _deg_kernel()(srcp, dstp)

    hp = jnp.concatenate([h, jnp.zeros((NP - N, D), jnp.float32)])
    x = _mm(hp, W_emb, b_emb.reshape(1, D))
    hx = _mms(x, dg, Wg[0])

    out = None
    for l in range(NL):
        parts = _seg_kernel()(hx, srcg, dstp)
        if l < NL - 1:
            x = pl.pallas_call(
                _t1a_body,
                out_shape=_sds((NP, D)),
            )(parts, x, dg, bg[l].reshape(1, D),
              gamma[l].reshape(1, D), beta[l].reshape(1, D))
            hx = _mms(x, dg, Wg[l + 1])
        else:
            out = pl.pallas_call(
                _tf_body,
                out_shape=_sds((1, NCLS)),
            )(parts, x, dg, bg[l].reshape(1, D), gamma[l].reshape(1, D),
              beta[l].reshape(1, D), W0, b0.reshape(1, D // 2),
              W1, b1.reshape(1, D // 4), W2, b2.reshape(1, NCLS))
    return out
